# baseline (speedup 1.0000x reference)
import functools
import jax
import jax.numpy as jnp
from jax import lax
from jax.experimental import pallas as pl
from jax.experimental.pallas import tpu as pltpu
from jax.experimental.pallas import tpu_sc as plsc

_NC, _NS = 2, 16
_NW = _NC * _NS
_CH = 8


def kernel(x, pos_table):
    B, S, D = x.shape
    pos = pos_table[:S]
    x2 = x.reshape(B * S, D)
    rows_per_w = S // _NW
    n_ch = rows_per_w // _CH
    mesh = plsc.VectorSubcoreMesh(core_axis_name="c", subcore_axis_name="s")

    @functools.partial(
        pl.kernel,
        out_type=jax.ShapeDtypeStruct((B * S, D), x.dtype),
        mesh=mesh,
        scratch_types=[
            pltpu.VMEM((_CH, D), jnp.float32),
            pltpu.VMEM((_CH, D), jnp.float32),
            pltpu.VMEM((B * _CH, D), jnp.float32),
            pltpu.VMEM((B * _CH, D), jnp.float32),
            pltpu.SemaphoreType.DMA,
            pltpu.SemaphoreType.DMA,
            pltpu.SemaphoreType.DMA,
            pltpu.SemaphoreType.DMA,
            pltpu.SemaphoreType.DMA,
            pltpu.SemaphoreType.DMA,
        ],
    )
    def sc_add(x_hbm, pos_hbm, out_hbm, pb0, pb1, xb0, xb1,
               ps0, ps1, xs0, xs1, os0, os1):
        wid = lax.axis_index("s") * _NC + lax.axis_index("c")
        s0 = wid * rows_per_w
        pb = (pb0, pb1)
        xb = (xb0, xb1)
        ps = (ps0, ps1)
        xs = (xs0, xs1)
        osm = (os0, os1)

        def fire_inputs(k, p):
            srow = s0 + k * _CH
            pltpu.async_copy(pos_hbm.at[pl.ds(srow, _CH)], pb[p], ps[p])
            for b in range(B):
                pltpu.async_copy(
                    x_hbm.at[pl.ds(b * S + srow, _CH)],
                    xb[p].at[pl.ds(b * _CH, _CH)],
                    xs[p],
                )

        def drain(sem, dst):
            pltpu.make_async_copy(x_hbm.at[pl.ds(0, dst.shape[0])], dst, sem).wait()

        fire_inputs(0, 0)

        def step(kk, carry):
            for p in range(2):
                k = 2 * kk + p
                np_ = 1 - p

                @pl.when(k >= 1)
                def _():
                    drain(osm[np_], xb[np_])

                @pl.when(k + 1 < n_ch)
                def _():
                    fire_inputs(k + 1, np_)

                drain(ps[p], pb[p])
                drain(xs[p], xb[p])

                @plsc.parallel_loop(0, D // 16, unroll=2)
                def col(c):
                    sl = pl.ds(c * 16, 16)
                    for r in range(_CH):
                        pv = pb[p][r, sl]
                        for b in range(B):
                            xb[p][b * _CH + r, sl] = xb[p][b * _CH + r, sl] + pv

                srow = s0 + k * _CH
                for b in range(B):
                    pltpu.async_copy(
                        xb[p].at[pl.ds(b * _CH, _CH)],
                        out_hbm.at[pl.ds(b * S + srow, _CH)],
                        osm[p],
                    )
            return carry

        lax.fori_loop(0, n_ch // 2, step, 0)
        drain(osm[1], xb[1])

    out = sc_add(x2, pos)
    return out.reshape(B, S, D)


# speedup vs baseline: 1.0227x; 1.0227x over previous
import functools
import jax
import jax.numpy as jnp
from jax import lax
from jax.experimental import pallas as pl
from jax.experimental.pallas import tpu as pltpu
from jax.experimental.pallas import tpu_sc as plsc

_NC, _NS = 2, 16
_NW = _NC * _NS
_CH = 8


def kernel(x, pos_table):
    B, S, D = x.shape
    pos = pos_table[:S]
    rows_per_w = S // _NW
    n_ch = rows_per_w // _CH
    mesh = plsc.VectorSubcoreMesh(core_axis_name="c", subcore_axis_name="s")

    @functools.partial(
        pl.kernel,
        out_type=jax.ShapeDtypeStruct((B, S, D), x.dtype),
        mesh=mesh,
        scratch_types=[
            pltpu.VMEM((_CH, D), jnp.float32),
            pltpu.VMEM((_CH, D), jnp.float32),
            pltpu.VMEM((B, _CH, D), jnp.float32),
            pltpu.VMEM((B, _CH, D), jnp.float32),
            pltpu.SemaphoreType.DMA,
            pltpu.SemaphoreType.DMA,
            pltpu.SemaphoreType.DMA,
            pltpu.SemaphoreType.DMA,
            pltpu.SemaphoreType.DMA,
            pltpu.SemaphoreType.DMA,
        ],
    )
    def sc_add(x_hbm, pos_hbm, out_hbm, pb0, pb1, xb0, xb1,
               ps0, ps1, xs0, xs1, os0, os1):
        wid = lax.axis_index("s") * _NC + lax.axis_index("c")
        s0 = wid * rows_per_w
        pb = (pb0, pb1)
        xb = (xb0, xb1)
        ps = (ps0, ps1)
        xs = (xs0, xs1)
        osm = (os0, os1)

        def fire_inputs(k, p):
            srow = s0 + k * _CH
            pltpu.async_copy(pos_hbm.at[pl.ds(srow, _CH)], pb[p], ps[p])
            pltpu.async_copy(x_hbm.at[:, pl.ds(srow, _CH)], xb[p], xs[p])

        def drain(sem, dst, src):
            pltpu.make_async_copy(src, dst, sem).wait()

        fire_inputs(0, 0)

        def step(kk, carry):
            for p in range(2):
                k = 2 * kk + p
                np_ = 1 - p

                @pl.when(k >= 1)
                def _():
                    drain(osm[np_], xb[np_], x_hbm.at[:, pl.ds(0, _CH)])

                @pl.when(k + 1 < n_ch)
                def _():
                    fire_inputs(k + 1, np_)

                drain(ps[p], pb[p], pos_hbm.at[pl.ds(0, _CH)])
                drain(xs[p], xb[p], x_hbm.at[:, pl.ds(0, _CH)])

                @plsc.parallel_loop(0, D // 16)
                def col(c):
                    sl = pl.ds(c * 16, 16)
                    for r in range(_CH):
                        pv = pb[p][r, sl]
                        for b in range(B):
                            xb[p][b, r, sl] = xb[p][b, r, sl] + pv

                srow = s0 + k * _CH
                pltpu.async_copy(xb[p], out_hbm.at[:, pl.ds(srow, _CH)], osm[p])
            return carry

        lax.fori_loop(0, n_ch // 2, step, 0)
        drain(osm[1], xb[1], x_hbm.at[:, pl.ds(0, _CH)])

    return sc_add(x, pos)


# speedup vs baseline: 1.0250x; 1.0023x over previous
import functools
import jax
import jax.numpy as jnp
from jax import lax
from jax.experimental import pallas as pl
from jax.experimental.pallas import tpu as pltpu
from jax.experimental.pallas import tpu_sc as plsc

_NC, _NS = 2, 16
_NW = _NC * _NS
_CH = 8
_NPH = 3


def kernel(x, pos_table):
    B, S, D = x.shape
    pos = pos_table[:S]
    rows_per_w = S // _NW
    n_ch = rows_per_w // _CH
    n_main = (n_ch // _NPH) * _NPH
    mesh = plsc.VectorSubcoreMesh(core_axis_name="c", subcore_axis_name="s")

    @functools.partial(
        pl.kernel,
        out_type=jax.ShapeDtypeStruct((B, S, D), x.dtype),
        mesh=mesh,
        scratch_types=(
            [pltpu.VMEM((_CH, D), jnp.float32) for _ in range(_NPH)]
            + [pltpu.VMEM((B, _CH, D), jnp.float32) for _ in range(_NPH)]
            + [pltpu.SemaphoreType.DMA] * (3 * _NPH)
        ),
    )
    def sc_add(x_hbm, pos_hbm, out_hbm, *bufs):
        pb = bufs[0:_NPH]
        xb = bufs[_NPH:2 * _NPH]
        ps = bufs[2 * _NPH:3 * _NPH]
        xs = bufs[3 * _NPH:4 * _NPH]
        osm = bufs[4 * _NPH:5 * _NPH]
        wid = lax.axis_index("s") * _NC + lax.axis_index("c")
        s0 = wid * rows_per_w

        def fire_inputs(k, p):
            srow = s0 + k * _CH
            pltpu.async_copy(pos_hbm.at[pl.ds(srow, _CH)], pb[p], ps[p])
            pltpu.async_copy(x_hbm.at[:, pl.ds(srow, _CH)], xb[p], xs[p])

        def drain(sem, dst, src):
            pltpu.make_async_copy(src, dst, sem).wait()

        def drain_outs(p):
            drain(osm[p], xb[p], x_hbm.at[:, pl.ds(0, _CH)])

        def do_step(k, p, static_k=None):
            np1 = (p + 1) % _NPH

            def _drain_prev():
                drain_outs(np1)

            def _fire_next():
                fire_inputs(k + 1, np1)

            if static_k is None:
                pl.when(k >= _NPH - 1)(_drain_prev)
                pl.when(k + 1 < n_ch)(_fire_next)
            else:
                if static_k >= _NPH - 1:
                    _drain_prev()
                if static_k + 1 < n_ch:
                    _fire_next()

            drain(ps[p], pb[p], pos_hbm.at[pl.ds(0, _CH)])
            drain(xs[p], xb[p], x_hbm.at[:, pl.ds(0, _CH)])

            @plsc.parallel_loop(0, D // 16)
            def col(c):
                sl = pl.ds(c * 16, 16)
                for r in range(_CH):
                    pv = pb[p][r, sl]
                    for b in range(B):
                        xb[p][b, r, sl] = xb[p][b, r, sl] + pv

            srow = s0 + k * _CH
            pltpu.async_copy(xb[p], out_hbm.at[:, pl.ds(srow, _CH)], osm[p])

        fire_inputs(0, 0)

        def step(kk, carry):
            for j in range(_NPH):
                do_step(_NPH * kk + j, j)
            return carry

        lax.fori_loop(0, n_main // _NPH, step, 0)
        for k in range(n_main, n_ch):
            do_step(k, k % _NPH, static_k=k)
        for k in range(max(n_ch - (_NPH - 1), 0), n_ch):
            drain_outs(k % _NPH)

    return sc_add(x, pos)
